# baseline (speedup 1.0000x reference)
import functools

import jax
import jax.numpy as jnp
import numpy as np
from jax import lax
from jax.experimental import pallas as pl
from jax.experimental.pallas import tpu as pltpu
from jax.experimental.pallas import tpu_sc as plsc


def _interleave_perm(width):
    k = np.arange(16)
    blk = np.empty(32, np.int64)
    blk[2 * k] = k
    blk[2 * k + 1] = 16 + k
    return np.concatenate([32 * g + blk for g in range(width // 32)])


def _edge_linear_body(ea_ref, wl_ref, bl_ref, out_ref):
    out_ref[...] = (
        jnp.dot(ea_ref[...], wl_ref[...], preferred_element_type=jnp.float32)
        + bl_ref[...]
    ).astype(jnp.bfloat16)


def _edge_linear(edge_attr, Wlin, blin, H):
    E, D = edge_attr.shape
    BE = 1600
    nb = E // BE
    grid = (2, nb)
    return pl.pallas_call(
        _edge_linear_body,
        grid=grid,
        in_specs=[
            pl.BlockSpec((BE, D), lambda c, i: (i, 0)),
            pl.BlockSpec((D, H), lambda c, i: (0, c)),
            pl.BlockSpec((1, H), lambda c, i: (0, c)),
        ],
        out_specs=pl.BlockSpec((BE, H), lambda c, i: (c * nb + i, 0)),
        out_shape=jax.ShapeDtypeStruct((2 * E, H), jnp.bfloat16),
    )(edge_attr, Wlin, blin.reshape(1, -1))


def _sc_aggregate(xh, eh, ei4, N, E, H, K, NB=3):
    NS = 16
    C = E // K
    cpt = C // NS
    NI = 3 * NB
    n_ring = (cpt // NI) * NI
    G = H // 16
    B8 = (N // NS) // 8 * 8
    REM = N - B8 * NS
    nz_full, nz_tail = B8 // K, B8 % K
    assert REM % 8 == 0 and REM <= K and nz_tail % 8 == 0

    mesh = plsc.VectorSubcoreMesh(core_axis_name="c", subcore_axis_name="s")

    W = H // 2
    scratch = (
        [pltpu.VMEM((2, K), jnp.int32) for _ in range(NI)]
        + [pltpu.VMEM((K, 2 * W), jnp.int32) for _ in range(NB)]
        + [pltpu.VMEM((K, W), jnp.int32) for _ in range(NB)]
        + [pltpu.VMEM((K, H), jnp.float32) for _ in range(NB)]
        + [pltpu.VMEM_SHARED((N, H), jnp.float32)]
        + [pltpu.SemaphoreType.DMA for _ in range(NI + 3 * NB)]
    )

    @functools.partial(
        pl.kernel,
        out_type=jax.ShapeDtypeStruct((2 * N, H), jnp.float32),
        mesh=mesh,
        scratch_types=scratch,
    )
    def body(xh_hbm, eh_hbm, ei_hbm, out_hbm, *refs):
        idx = refs[0:NI]
        rows = refs[NI:NI + NB]
        ebuf = refs[NI + NB:NI + 2 * NB]
        msg = refs[NI + 2 * NB:NI + 3 * NB]
        acc = refs[NI + 3 * NB]
        si = refs[NI + 3 * NB + 1:2 * NI + 3 * NB + 1]
        sg = refs[2 * NI + 3 * NB + 1:2 * NI + 4 * NB + 1]
        se = refs[2 * NI + 4 * NB + 1:2 * NI + 5 * NB + 1]
        ssc = refs[2 * NI + 5 * NB + 1:2 * NI + 6 * NB + 1]

        cid = lax.axis_index("c")
        sid = lax.axis_index("s")
        base_chunk = sid * cpt

        def issue_idx(ib, t):
            pltpu.async_copy(ei_hbm.at[base_chunk + t], idx[ib], si[ib])

        def wait_idx(ib):
            pltpu.make_async_copy(
                ei_hbm.at[0], idx[ib], si[ib]).wait()

        def issue_fetch(b, ib, t):
            pltpu.async_copy(xh_hbm.at[idx[ib].at[0]], rows[b], sg[b])
            erow = pl.multiple_of((cid * C + base_chunk + t) * K, 8)
            pltpu.async_copy(eh_hbm.at[pl.ds(erow, K)], ebuf[b], se[b])

        def wait_fetch(b):
            pltpu.make_async_copy(
                xh_hbm.at[pl.ds(0, K)], rows[b], sg[b]).wait()
            pltpu.make_async_copy(
                eh_hbm.at[pl.ds(0, K)], ebuf[b], se[b]).wait()

        himask = jnp.int32(-65536)

        def compute(b):
            def row(j, c):
                for g in range(G // 2):
                    xw = rows[b][j, pl.ds(
                        pl.multiple_of(cid * W + g * 16, 16), 16)]
                    ew = ebuf[b][j, pl.ds(g * 16, 16)]
                    xa = lax.bitcast_convert_type(xw << 16, jnp.float32)
                    xb = lax.bitcast_convert_type(xw & himask, jnp.float32)
                    ea = lax.bitcast_convert_type(ew << 16, jnp.float32)
                    eb = lax.bitcast_convert_type(ew & himask, jnp.float32)
                    msg[b][j, pl.ds(g * 32, 16)] = jnp.maximum(xa + ea, 0.0)
                    msg[b][j, pl.ds(g * 32 + 16, 16)] = jnp.maximum(
                        xb + eb, 0.0)
                return c

            lax.fori_loop(0, K, row, 0)

        def issue_scatter(b, ib):
            pltpu.async_copy(msg[b], acc.at[idx[ib].at[1]], ssc[b], add=True)

        def wait_scatter(b):
            pltpu.make_async_copy(
                msg[b], acc.at[idx[0].at[1]], ssc[b]).wait()

        for j in range(min(2 * NB, cpt)):
            issue_idx(j, j)

        def zero_row(j, c):
            for g in range(G):
                msg[0][j, pl.ds(g * 16, 16)] = jnp.zeros((16,), jnp.float32)
            return c

        lax.fori_loop(0, K, zero_row, 0)
        for i in range(nz_full):
            pltpu.sync_copy(
                msg[0], acc.at[pl.ds(pl.multiple_of(sid * B8 + i * K, 8), K)])
        if nz_tail:
            pltpu.sync_copy(
                msg[0].at[pl.ds(0, nz_tail)],
                acc.at[pl.ds(pl.multiple_of(sid * B8 + nz_full * K, 8),
                             nz_tail)])
        if REM:
            @pl.when(sid == NS - 1)
            def _():
                pltpu.sync_copy(msg[0].at[pl.ds(0, REM)],
                                acc.at[pl.ds(N - REM, REM)])

        for b in range(NB):
            wait_idx(b)
            issue_fetch(b, b, b)
        plsc.subcore_barrier()

        def ring(q, c):
            t0 = q * NI
            for j in range(NI):
                t = t0 + j
                b = j % NB
                wait_fetch(b)

                @pl.when(t >= NB)
                def _():
                    wait_scatter(b)

                compute(b)
                issue_scatter(b, j)
                tn = t + 2 * NB

                @pl.when(tn < cpt)
                def _():
                    issue_idx((j + 2 * NB) % NI, tn)

                tf = t + NB

                @pl.when(tf < cpt)
                def _():
                    wait_idx((j + NB) % NI)
                    issue_fetch(b, (j + NB) % NI, tf)
            return c

        lax.fori_loop(0, n_ring // NI, ring, 0)
        for t in range(n_ring, cpt):
            j = t % NI
            b = j % NB
            wait_fetch(b)
            if t >= NB:
                wait_scatter(b)
            compute(b)
            issue_scatter(b, j)
            tn = t + 2 * NB
            if tn < cpt:
                issue_idx((j + 2 * NB) % NI, tn)
            tf = t + NB
            if tf < cpt:
                wait_idx((j + NB) % NI)
                issue_fetch(b, (j + NB) % NI, tf)
        for b in range(min(NB, cpt)):
            wait_scatter(b)
        plsc.subcore_barrier()

        r0 = pl.multiple_of(sid * B8, 8)
        pltpu.sync_copy(
            acc.at[pl.ds(r0, B8)],
            out_hbm.at[pl.ds(pl.multiple_of(cid * N + sid * B8, 8), B8)],
        )
        if REM:
            @pl.when(sid == NS - 1)
            def _():
                pltpu.sync_copy(
                    acc.at[pl.ds(N - REM, REM)],
                    out_hbm.at[pl.ds(pl.multiple_of(cid * N + N - REM, 8),
                                     REM)],
                )

    return body(xh, eh, ei4)


def _mlp_bn_body(x_ref, ah_ref, w1_ref, b1_ref, w2_ref, b2_ref,
                 eps_ref, gamma_ref, beta_ref, out_ref):
    n = x_ref.shape[0]
    x = x_ref[...]
    aggr = jnp.concatenate([ah_ref[:n, :], ah_ref[n:, :]], axis=1)
    h = (1.0 + eps_ref[0, 0]) * x + aggr
    h1 = jnp.maximum(
        jnp.dot(h, w1_ref[...], preferred_element_type=jnp.float32)
        + b1_ref[...], 0.0)
    h2 = (jnp.dot(h1, w2_ref[...], preferred_element_type=jnp.float32)
          + b2_ref[...])
    mean = jnp.mean(h2, axis=0, keepdims=True)
    var = jnp.mean((h2 - mean) ** 2, axis=0, keepdims=True)
    hn = (h2 - mean) * lax.rsqrt(var + 1e-5) * gamma_ref[...] + beta_ref[...]
    out_ref[...] = jnp.maximum(hn + x, 0.0)


def _mlp_bn(x, aggr2, W1, b1, W2, b2, eps, gamma, beta):
    N, F = x.shape
    return pl.pallas_call(
        _mlp_bn_body,
        out_shape=jax.ShapeDtypeStruct((N, F), jnp.float32),
    )(x, aggr2, W1, b1.reshape(1, -1), W2, b2.reshape(1, -1),
      eps.reshape(1, 1), gamma.reshape(1, -1), beta.reshape(1, -1))


def kernel(x, edge_index, edge_attr, Wlin, blin, W1, b1, W2, b2,
           eps, gamma, beta):
    N, F = x.shape
    E = edge_index.shape[1]
    H = F // 2
    K = 40
    src = edge_index[0]
    dst = edge_index[1]
    srcr = src.reshape(E // K, K)
    dstr = dst.reshape(E // K, K)
    ei3 = jnp.stack([srcr, dstr], axis=1)

    inv_half = _interleave_perm(H)
    colperm = np.concatenate([inv_half, H + inv_half])
    eh = _edge_linear(edge_attr, Wlin[:, colperm], blin[colperm], H)
    eh32 = lax.bitcast_convert_type(
        eh.reshape(2 * E, H // 2, 2), jnp.int32)
    xp = (x[:, colperm]).astype(jnp.bfloat16)
    x32 = lax.bitcast_convert_type(xp.reshape(N, F // 2, 2), jnp.int32)
    aggr2 = _sc_aggregate(x32, eh32, ei3, N, E, H, K, NB=3)
    return _mlp_bn(x, aggr2, W1, b1, W2, b2, eps, gamma, beta)


# speedup vs baseline: 1.0204x; 1.0204x over previous
import functools

import jax
import jax.numpy as jnp
import numpy as np
from jax import lax
from jax.experimental import pallas as pl
from jax.experimental.pallas import tpu as pltpu
from jax.experimental.pallas import tpu_sc as plsc


def _unperm_matrix(F):
    k = np.arange(16)
    fwd = np.empty(32, np.int64)
    fwd[k] = 2 * k
    fwd[16 + k] = 2 * k + 1
    half = np.concatenate([32 * g + fwd for g in range(F // 2 // 32)])
    featof = np.concatenate([half, F // 2 + half])
    P = np.zeros((F, F), np.float32)
    P[np.arange(F), featof] = 1.0
    return P


def _edge_linear_body(ea_ref, wl_ref, bl_ref, out_ref):
    out_ref[...] = (
        jnp.dot(ea_ref[...], wl_ref[...], preferred_element_type=jnp.float32)
        + bl_ref[...]
    ).astype(jnp.bfloat16)


def _edge_linear(edge_attr, Wlin, blin, H):
    E, D = edge_attr.shape
    BE = 1600
    nb = E // BE
    grid = (2, nb)
    return pl.pallas_call(
        _edge_linear_body,
        grid=grid,
        in_specs=[
            pl.BlockSpec((BE, D), lambda c, i: (i, 0)),
            pl.BlockSpec((D, H), lambda c, i: (0, c)),
            pl.BlockSpec((1, H), lambda c, i: (0, c)),
        ],
        out_specs=pl.BlockSpec((BE, H), lambda c, i: (c * nb + i, 0)),
        out_shape=jax.ShapeDtypeStruct((2 * E, H), jnp.bfloat16),
    )(edge_attr, Wlin, blin.reshape(1, -1))


def _sc_aggregate(xh, eh, ei4, N, E, H, K, NB=3):
    NS = 16
    C = E // K
    cpt = C // NS
    NI = 3 * NB
    n_ring = (cpt // NI) * NI
    G = H // 16
    B8 = (N // NS) // 8 * 8
    REM = N - B8 * NS
    nz_full, nz_tail = B8 // K, B8 % K
    assert REM % 8 == 0 and REM <= K and nz_tail % 8 == 0

    mesh = plsc.VectorSubcoreMesh(core_axis_name="c", subcore_axis_name="s")

    W = H // 2
    scratch = (
        [pltpu.VMEM((2, K), jnp.int32) for _ in range(NI)]
        + [pltpu.VMEM((K, 2 * W), jnp.int32) for _ in range(NB)]
        + [pltpu.VMEM((K, W), jnp.int32) for _ in range(NB)]
        + [pltpu.VMEM((K, H), jnp.float32) for _ in range(NB)]
        + [pltpu.VMEM_SHARED((N, H), jnp.float32)]
        + [pltpu.SemaphoreType.DMA for _ in range(NI + 3 * NB)]
    )

    @functools.partial(
        pl.kernel,
        out_type=jax.ShapeDtypeStruct((2 * N, H), jnp.float32),
        mesh=mesh,
        scratch_types=scratch,
    )
    def body(xh_hbm, eh_hbm, ei_hbm, out_hbm, *refs):
        idx = refs[0:NI]
        rows = refs[NI:NI + NB]
        ebuf = refs[NI + NB:NI + 2 * NB]
        msg = refs[NI + 2 * NB:NI + 3 * NB]
        acc = refs[NI + 3 * NB]
        si = refs[NI + 3 * NB + 1:2 * NI + 3 * NB + 1]
        sg = refs[2 * NI + 3 * NB + 1:2 * NI + 4 * NB + 1]
        se = refs[2 * NI + 4 * NB + 1:2 * NI + 5 * NB + 1]
        ssc = refs[2 * NI + 5 * NB + 1:2 * NI + 6 * NB + 1]

        cid = lax.axis_index("c")
        sid = lax.axis_index("s")
        base_chunk = sid * cpt

        def issue_idx(ib, t):
            pltpu.async_copy(ei_hbm.at[base_chunk + t], idx[ib], si[ib])

        def wait_idx(ib):
            pltpu.make_async_copy(
                ei_hbm.at[0], idx[ib], si[ib]).wait()

        def issue_fetch(b, ib, t):
            pltpu.async_copy(xh_hbm.at[idx[ib].at[0]], rows[b], sg[b])
            erow = pl.multiple_of((cid * C + base_chunk + t) * K, 8)
            pltpu.async_copy(eh_hbm.at[pl.ds(erow, K)], ebuf[b], se[b])

        def wait_fetch(b):
            pltpu.make_async_copy(
                xh_hbm.at[pl.ds(0, K)], rows[b], sg[b]).wait()
            pltpu.make_async_copy(
                eh_hbm.at[pl.ds(0, K)], ebuf[b], se[b]).wait()

        himask = jnp.int32(-65536)

        def compute(b):
            def row(j, c):
                for g in range(G // 2):
                    xw = rows[b][j, pl.ds(
                        pl.multiple_of(cid * W + g * 16, 16), 16)]
                    ew = ebuf[b][j, pl.ds(g * 16, 16)]
                    xa = lax.bitcast_convert_type(xw << 16, jnp.float32)
                    xb = lax.bitcast_convert_type(xw & himask, jnp.float32)
                    ea = lax.bitcast_convert_type(ew << 16, jnp.float32)
                    eb = lax.bitcast_convert_type(ew & himask, jnp.float32)
                    msg[b][j, pl.ds(g * 32, 16)] = jnp.maximum(xa + ea, 0.0)
                    msg[b][j, pl.ds(g * 32 + 16, 16)] = jnp.maximum(
                        xb + eb, 0.0)
                return c

            lax.fori_loop(0, K, row, 0)

        def issue_scatter(b, ib):
            pltpu.async_copy(msg[b], acc.at[idx[ib].at[1]], ssc[b], add=True)

        def wait_scatter(b):
            pltpu.make_async_copy(
                msg[b], acc.at[idx[0].at[1]], ssc[b]).wait()

        for j in range(min(2 * NB, cpt)):
            issue_idx(j, j)

        def zero_row(j, c):
            for g in range(G):
                msg[0][j, pl.ds(g * 16, 16)] = jnp.zeros((16,), jnp.float32)
            return c

        lax.fori_loop(0, K, zero_row, 0)
        for i in range(nz_full):
            pltpu.sync_copy(
                msg[0], acc.at[pl.ds(pl.multiple_of(sid * B8 + i * K, 8), K)])
        if nz_tail:
            pltpu.sync_copy(
                msg[0].at[pl.ds(0, nz_tail)],
                acc.at[pl.ds(pl.multiple_of(sid * B8 + nz_full * K, 8),
                             nz_tail)])
        if REM:
            @pl.when(sid == NS - 1)
            def _():
                pltpu.sync_copy(msg[0].at[pl.ds(0, REM)],
                                acc.at[pl.ds(N - REM, REM)])

        for b in range(NB):
            wait_idx(b)
            issue_fetch(b, b, b)
        plsc.subcore_barrier()

        def ring(q, c):
            t0 = q * NI
            for j in range(NI):
                t = t0 + j
                b = j % NB
                wait_fetch(b)

                @pl.when(t >= NB)
                def _():
                    wait_scatter(b)

                compute(b)
                issue_scatter(b, j)
                tn = t + 2 * NB

                @pl.when(tn < cpt)
                def _():
                    issue_idx((j + 2 * NB) % NI, tn)

                tf = t + NB

                @pl.when(tf < cpt)
                def _():
                    wait_idx((j + NB) % NI)
                    issue_fetch(b, (j + NB) % NI, tf)
            return c

        lax.fori_loop(0, n_ring // NI, ring, 0)
        for t in range(n_ring, cpt):
            j = t % NI
            b = j % NB
            wait_fetch(b)
            if t >= NB:
                wait_scatter(b)
            compute(b)
            issue_scatter(b, j)
            tn = t + 2 * NB
            if tn < cpt:
                issue_idx((j + 2 * NB) % NI, tn)
            tf = t + NB
            if tf < cpt:
                wait_idx((j + NB) % NI)
                issue_fetch(b, (j + NB) % NI, tf)
        for b in range(min(NB, cpt)):
            wait_scatter(b)
        plsc.subcore_barrier()

        r0 = pl.multiple_of(sid * B8, 8)
        pltpu.sync_copy(
            acc.at[pl.ds(r0, B8)],
            out_hbm.at[pl.ds(pl.multiple_of(cid * N + sid * B8, 8), B8)],
        )
        if REM:
            @pl.when(sid == NS - 1)
            def _():
                pltpu.sync_copy(
                    acc.at[pl.ds(N - REM, REM)],
                    out_hbm.at[pl.ds(pl.multiple_of(cid * N + N - REM, 8),
                                     REM)],
                )

    return body(xh, eh, ei4)


def _mlp_bn_body(x_ref, ah_ref, pun_ref, w1_ref, b1_ref, w2_ref, b2_ref,
                 eps_ref, gamma_ref, beta_ref, out_ref):
    n = x_ref.shape[0]
    x = x_ref[...]
    aggr = jnp.dot(
        jnp.concatenate([ah_ref[:n, :], ah_ref[n:, :]], axis=1),
        pun_ref[...], preferred_element_type=jnp.float32)
    h = (1.0 + eps_ref[0, 0]) * x + aggr
    h1 = jnp.maximum(
        jnp.dot(h, w1_ref[...], preferred_element_type=jnp.float32)
        + b1_ref[...], 0.0)
    h2 = (jnp.dot(h1, w2_ref[...], preferred_element_type=jnp.float32)
          + b2_ref[...])
    mean = jnp.mean(h2, axis=0, keepdims=True)
    var = jnp.mean((h2 - mean) ** 2, axis=0, keepdims=True)
    hn = (h2 - mean) * lax.rsqrt(var + 1e-5) * gamma_ref[...] + beta_ref[...]
    out_ref[...] = jnp.maximum(hn + x, 0.0)


def _mlp_bn(x, aggr2, pun, W1, b1, W2, b2, eps, gamma, beta):
    N, F = x.shape
    return pl.pallas_call(
        _mlp_bn_body,
        out_shape=jax.ShapeDtypeStruct((N, F), jnp.float32),
    )(x, aggr2, pun, W1, b1.reshape(1, -1), W2, b2.reshape(1, -1),
      eps.reshape(1, 1), gamma.reshape(1, -1), beta.reshape(1, -1))


def kernel(x, edge_index, edge_attr, Wlin, blin, W1, b1, W2, b2,
           eps, gamma, beta):
    N, F = x.shape
    E = edge_index.shape[1]
    H = F // 2
    K = 40
    src = edge_index[0]
    dst = edge_index[1]
    srcr = src.reshape(E // K, K)
    dstr = dst.reshape(E // K, K)
    ei3 = jnp.stack([srcr, dstr], axis=1)

    eh = _edge_linear(edge_attr, Wlin, blin, H)
    eh32 = lax.bitcast_convert_type(
        eh.reshape(2 * E, H // 2, 2), jnp.int32)
    x32 = lax.bitcast_convert_type(
        x.astype(jnp.bfloat16).reshape(N, F // 2, 2), jnp.int32)
    aggr2 = _sc_aggregate(x32, eh32, ei3, N, E, H, K, NB=3)
    pun = jnp.asarray(_unperm_matrix(F))
    return _mlp_bn(x, aggr2, pun, W1, b1, W2, b2, eps, gamma, beta)


# speedup vs baseline: 2.1680x; 2.1247x over previous
import functools

import jax
import jax.numpy as jnp
import numpy as np
from jax import lax
from jax.experimental import pallas as pl
from jax.experimental.pallas import tpu as pltpu
from jax.experimental.pallas import tpu_sc as plsc


def _unperm_matrix(F):
    k = np.arange(16)
    fwd = np.empty(32, np.int64)
    fwd[k] = 2 * k
    fwd[16 + k] = 2 * k + 1
    half = np.concatenate([32 * g + fwd for g in range(F // 2 // 32)])
    featof = np.concatenate([half, F // 2 + half])
    P = np.zeros((F, F), np.float32)
    P[np.arange(F), featof] = 1.0
    return P


def _edge_linear_body(ea_ref, wle_ref, wlo_ref, ble_ref, blo_ref, out_ref):
    ea = ea_ref[...]
    ev = (jnp.dot(ea, wle_ref[0], preferred_element_type=jnp.float32)
          + ble_ref[0])
    od = (jnp.dot(ea, wlo_ref[0], preferred_element_type=jnp.float32)
          + blo_ref[0])
    evb = lax.bitcast_convert_type(ev.astype(jnp.bfloat16),
                                   jnp.uint16).astype(jnp.int32)
    odb = lax.bitcast_convert_type(od.astype(jnp.bfloat16),
                                   jnp.uint16).astype(jnp.int32)
    out_ref[...] = evb | (odb << 16)


def _edge_linear(edge_attr, Wlin, blin, H):
    E, D = edge_attr.shape
    W = H // 2
    BE = 1600
    nb = E // BE
    grid = (2, nb)
    return pl.pallas_call(
        _edge_linear_body,
        grid=grid,
        in_specs=[
            pl.BlockSpec((BE, D), lambda c, i: (i, 0)),
            pl.BlockSpec((1, D, W), lambda c, i: (c, 0, 0)),
            pl.BlockSpec((1, D, W), lambda c, i: (c, 0, 0)),
            pl.BlockSpec((1, 1, W), lambda c, i: (c, 0, 0)),
            pl.BlockSpec((1, 1, W), lambda c, i: (c, 0, 0)),
        ],
        out_specs=pl.BlockSpec((BE, W), lambda c, i: (c * nb + i, 0)),
        out_shape=jax.ShapeDtypeStruct((2 * E, W), jnp.int32),
    )(edge_attr,
      Wlin[:, 0::2].reshape(D, 2, W).transpose(1, 0, 2),
      Wlin[:, 1::2].reshape(D, 2, W).transpose(1, 0, 2),
      blin[0::2].reshape(1, 2, W).transpose(1, 0, 2),
      blin[1::2].reshape(1, 2, W).transpose(1, 0, 2))


def _sc_aggregate(xh, eh, ei4, N, E, H, K, NB=3):
    NS = 16
    C = E // K
    cpt = C // NS
    NI = 3 * NB
    n_ring = (cpt // NI) * NI
    G = H // 16
    B8 = (N // NS) // 8 * 8
    REM = N - B8 * NS
    nz_full, nz_tail = B8 // K, B8 % K
    assert REM % 8 == 0 and REM <= K and nz_tail % 8 == 0

    mesh = plsc.VectorSubcoreMesh(core_axis_name="c", subcore_axis_name="s")

    W = H // 2
    scratch = (
        [pltpu.VMEM((2, K), jnp.int32) for _ in range(NI)]
        + [pltpu.VMEM((K, 2 * W), jnp.int32) for _ in range(NB)]
        + [pltpu.VMEM((K, W), jnp.int32) for _ in range(NB)]
        + [pltpu.VMEM((K, H), jnp.float32) for _ in range(NB)]
        + [pltpu.VMEM_SHARED((N, H), jnp.float32)]
        + [pltpu.SemaphoreType.DMA for _ in range(NI + 3 * NB)]
    )

    @functools.partial(
        pl.kernel,
        out_type=jax.ShapeDtypeStruct((2 * N, H), jnp.float32),
        mesh=mesh,
        scratch_types=scratch,
    )
    def body(xh_hbm, eh_hbm, ei_hbm, out_hbm, *refs):
        idx = refs[0:NI]
        rows = refs[NI:NI + NB]
        ebuf = refs[NI + NB:NI + 2 * NB]
        msg = refs[NI + 2 * NB:NI + 3 * NB]
        acc = refs[NI + 3 * NB]
        si = refs[NI + 3 * NB + 1:2 * NI + 3 * NB + 1]
        sg = refs[2 * NI + 3 * NB + 1:2 * NI + 4 * NB + 1]
        se = refs[2 * NI + 4 * NB + 1:2 * NI + 5 * NB + 1]
        ssc = refs[2 * NI + 5 * NB + 1:2 * NI + 6 * NB + 1]

        cid = lax.axis_index("c")
        sid = lax.axis_index("s")
        base_chunk = sid * cpt

        def issue_idx(ib, t):
            pltpu.async_copy(ei_hbm.at[base_chunk + t], idx[ib], si[ib])

        def wait_idx(ib):
            pltpu.make_async_copy(
                ei_hbm.at[0], idx[ib], si[ib]).wait()

        def issue_fetch(b, ib, t):
            pltpu.async_copy(xh_hbm.at[idx[ib].at[0]], rows[b], sg[b])
            erow = pl.multiple_of((cid * C + base_chunk + t) * K, 8)
            pltpu.async_copy(eh_hbm.at[pl.ds(erow, K)], ebuf[b], se[b])

        def wait_fetch(b):
            pltpu.make_async_copy(
                xh_hbm.at[pl.ds(0, K)], rows[b], sg[b]).wait()
            pltpu.make_async_copy(
                eh_hbm.at[pl.ds(0, K)], ebuf[b], se[b]).wait()

        himask = jnp.int32(-65536)

        def compute(b):
            def row(j, c):
                for g in range(G // 2):
                    xw = rows[b][j, pl.ds(
                        pl.multiple_of(cid * W + g * 16, 16), 16)]
                    ew = ebuf[b][j, pl.ds(g * 16, 16)]
                    xa = lax.bitcast_convert_type(xw << 16, jnp.float32)
                    xb = lax.bitcast_convert_type(xw & himask, jnp.float32)
                    ea = lax.bitcast_convert_type(ew << 16, jnp.float32)
                    eb = lax.bitcast_convert_type(ew & himask, jnp.float32)
                    msg[b][j, pl.ds(g * 32, 16)] = jnp.maximum(xa + ea, 0.0)
                    msg[b][j, pl.ds(g * 32 + 16, 16)] = jnp.maximum(
                        xb + eb, 0.0)
                return c

            lax.fori_loop(0, K, row, 0)

        def issue_scatter(b, ib):
            pltpu.async_copy(msg[b], acc.at[idx[ib].at[1]], ssc[b], add=True)

        def wait_scatter(b):
            pltpu.make_async_copy(
                msg[b], acc.at[idx[0].at[1]], ssc[b]).wait()

        for j in range(min(2 * NB, cpt)):
            issue_idx(j, j)

        def zero_row(j, c):
            for g in range(G):
                msg[0][j, pl.ds(g * 16, 16)] = jnp.zeros((16,), jnp.float32)
            return c

        lax.fori_loop(0, K, zero_row, 0)
        for i in range(nz_full):
            pltpu.sync_copy(
                msg[0], acc.at[pl.ds(pl.multiple_of(sid * B8 + i * K, 8), K)])
        if nz_tail:
            pltpu.sync_copy(
                msg[0].at[pl.ds(0, nz_tail)],
                acc.at[pl.ds(pl.multiple_of(sid * B8 + nz_full * K, 8),
                             nz_tail)])
        if REM:
            @pl.when(sid == NS - 1)
            def _():
                pltpu.sync_copy(msg[0].at[pl.ds(0, REM)],
                                acc.at[pl.ds(N - REM, REM)])

        for b in range(NB):
            wait_idx(b)
            issue_fetch(b, b, b)
        plsc.subcore_barrier()

        def ring(q, c):
            t0 = q * NI
            for j in range(NI):
                t = t0 + j
                b = j % NB
                wait_fetch(b)

                @pl.when(t >= NB)
                def _():
                    wait_scatter(b)

                compute(b)
                issue_scatter(b, j)
                tn = t + 2 * NB

                @pl.when(tn < cpt)
                def _():
                    issue_idx((j + 2 * NB) % NI, tn)

                tf = t + NB

                @pl.when(tf < cpt)
                def _():
                    wait_idx((j + NB) % NI)
                    issue_fetch(b, (j + NB) % NI, tf)
            return c

        lax.fori_loop(0, n_ring // NI, ring, 0)
        for t in range(n_ring, cpt):
            j = t % NI
            b = j % NB
            wait_fetch(b)
            if t >= NB:
                wait_scatter(b)
            compute(b)
            issue_scatter(b, j)
            tn = t + 2 * NB
            if tn < cpt:
                issue_idx((j + 2 * NB) % NI, tn)
            tf = t + NB
            if tf < cpt:
                wait_idx((j + NB) % NI)
                issue_fetch(b, (j + NB) % NI, tf)
        for b in range(min(NB, cpt)):
            wait_scatter(b)
        plsc.subcore_barrier()

        r0 = pl.multiple_of(sid * B8, 8)
        pltpu.sync_copy(
            acc.at[pl.ds(r0, B8)],
            out_hbm.at[pl.ds(pl.multiple_of(cid * N + sid * B8, 8), B8)],
        )
        if REM:
            @pl.when(sid == NS - 1)
            def _():
                pltpu.sync_copy(
                    acc.at[pl.ds(N - REM, REM)],
                    out_hbm.at[pl.ds(pl.multiple_of(cid * N + N - REM, 8),
                                     REM)],
                )

    return body(xh, eh, ei4)


def _mlp_bn_body(x_ref, ah_ref, pun_ref, w1_ref, b1_ref, w2_ref, b2_ref,
                 eps_ref, gamma_ref, beta_ref, out_ref):
    n = x_ref.shape[0]
    x = x_ref[...]
    aggr = jnp.dot(
        jnp.concatenate([ah_ref[:n, :], ah_ref[n:, :]], axis=1),
        pun_ref[...], preferred_element_type=jnp.float32)
    h = (1.0 + eps_ref[0, 0]) * x + aggr
    h1 = jnp.maximum(
        jnp.dot(h, w1_ref[...], preferred_element_type=jnp.float32)
        + b1_ref[...], 0.0)
    h2 = (jnp.dot(h1, w2_ref[...], preferred_element_type=jnp.float32)
          + b2_ref[...])
    mean = jnp.mean(h2, axis=0, keepdims=True)
    var = jnp.mean((h2 - mean) ** 2, axis=0, keepdims=True)
    hn = (h2 - mean) * lax.rsqrt(var + 1e-5) * gamma_ref[...] + beta_ref[...]
    out_ref[...] = jnp.maximum(hn + x, 0.0)


def _mlp_bn(x, aggr2, pun, W1, b1, W2, b2, eps, gamma, beta):
    N, F = x.shape
    return pl.pallas_call(
        _mlp_bn_body,
        out_shape=jax.ShapeDtypeStruct((N, F), jnp.float32),
    )(x, aggr2, pun, W1, b1.reshape(1, -1), W2, b2.reshape(1, -1),
      eps.reshape(1, 1), gamma.reshape(1, -1), beta.reshape(1, -1))


def kernel(x, edge_index, edge_attr, Wlin, blin, W1, b1, W2, b2,
           eps, gamma, beta):
    N, F = x.shape
    E = edge_index.shape[1]
    H = F // 2
    K = 40
    src = edge_index[0]
    dst = edge_index[1]
    srcr = src.reshape(E // K, K)
    dstr = dst.reshape(E // K, K)
    ei3 = jnp.stack([srcr, dstr], axis=1)

    eh32 = _edge_linear(edge_attr, Wlin, blin, H)
    x32 = lax.bitcast_convert_type(
        x.astype(jnp.bfloat16).reshape(N, F // 2, 2), jnp.int32)
    aggr2 = _sc_aggregate(x32, eh32, ei3, N, E, H, K, NB=3)
    pun = jnp.asarray(_unperm_matrix(F))
    return _mlp_bn(x, aggr2, pun, W1, b1, W2, b2, eps, gamma, beta)


# speedup vs baseline: 3.0445x; 1.4043x over previous
import functools

import jax
import jax.numpy as jnp
from jax import lax
from jax.experimental import pallas as pl
from jax.experimental.pallas import tpu as pltpu
from jax.experimental.pallas import tpu_sc as plsc


def _edge_linear_body(ea_ref, wl_ref, bl_ref, out_ref):
    out_ref[...] = (
        jnp.dot(ea_ref[...], wl_ref[0], preferred_element_type=jnp.float32)
        + bl_ref[0])


def _edge_linear(edge_attr, Wlin, blin, H):
    E, D = edge_attr.shape
    BE = 1600
    nb = E // BE
    grid = (2, nb)
    return pl.pallas_call(
        _edge_linear_body,
        grid=grid,
        in_specs=[
            pl.BlockSpec((BE, D), lambda c, i: (i, 0)),
            pl.BlockSpec((1, D, H), lambda c, i: (c, 0, 0)),
            pl.BlockSpec((1, 1, H), lambda c, i: (c, 0, 0)),
        ],
        out_specs=pl.BlockSpec((BE, H), lambda c, i: (c * nb + i, 0)),
        out_shape=jax.ShapeDtypeStruct((2 * E, H), jnp.float32),
    )(edge_attr,
      Wlin.reshape(D, 2, H).transpose(1, 0, 2),
      blin.reshape(1, 2, H).transpose(1, 0, 2))


def _sc_aggregate(xh, eh, ei4, N, E, H, K, NB=3):
    NS = 16
    C = E // K
    cpt = C // NS
    NI = 3 * NB
    n_ring = (cpt // NI) * NI
    G = H // 16
    B8 = (N // NS) // 8 * 8
    REM = N - B8 * NS
    nz_full, nz_tail = B8 // K, B8 % K
    assert REM % 8 == 0 and REM <= K and nz_tail % 8 == 0

    mesh = plsc.VectorSubcoreMesh(core_axis_name="c", subcore_axis_name="s")

    scratch = (
        [pltpu.VMEM((2, K), jnp.int32) for _ in range(NI)]
        + [pltpu.VMEM((K, H), jnp.float32) for _ in range(NB)]
        + [pltpu.VMEM((K, H), jnp.float32) for _ in range(NB)]
        + [pltpu.VMEM((K, H), jnp.float32) for _ in range(NB)]
        + [pltpu.VMEM_SHARED((N, H), jnp.float32)]
        + [pltpu.SemaphoreType.DMA for _ in range(NI + 3 * NB)]
    )

    @functools.partial(
        pl.kernel,
        out_type=jax.ShapeDtypeStruct((2 * N, H), jnp.float32),
        mesh=mesh,
        scratch_types=scratch,
    )
    def body(xh_hbm, eh_hbm, ei_hbm, out_hbm, *refs):
        idx = refs[0:NI]
        rows = refs[NI:NI + NB]
        ebuf = refs[NI + NB:NI + 2 * NB]
        msg = refs[NI + 2 * NB:NI + 3 * NB]
        acc = refs[NI + 3 * NB]
        si = refs[NI + 3 * NB + 1:2 * NI + 3 * NB + 1]
        sg = refs[2 * NI + 3 * NB + 1:2 * NI + 4 * NB + 1]
        se = refs[2 * NI + 4 * NB + 1:2 * NI + 5 * NB + 1]
        ssc = refs[2 * NI + 5 * NB + 1:2 * NI + 6 * NB + 1]

        cid = lax.axis_index("c")
        sid = lax.axis_index("s")
        base_chunk = cid * C + sid * cpt

        def issue_idx(ib, t):
            pltpu.async_copy(ei_hbm.at[base_chunk + t], idx[ib], si[ib])

        def wait_idx(ib):
            pltpu.make_async_copy(
                ei_hbm.at[0], idx[ib], si[ib]).wait()

        def issue_fetch(b, ib, t):
            pltpu.async_copy(xh_hbm.at[idx[ib].at[0]], rows[b], sg[b])
            erow = pl.multiple_of((base_chunk + t) * K, 8)
            pltpu.async_copy(eh_hbm.at[pl.ds(erow, K)], ebuf[b], se[b])

        def wait_fetch(b):
            pltpu.make_async_copy(
                xh_hbm.at[pl.ds(0, K)], rows[b], sg[b]).wait()
            pltpu.make_async_copy(
                eh_hbm.at[pl.ds(0, K)], ebuf[b], se[b]).wait()

        def compute(b):
            def row(j, c):
                for g in range(G):
                    xr = rows[b][j, pl.ds(g * 16, 16)]
                    er = ebuf[b][j, pl.ds(g * 16, 16)]
                    msg[b][j, pl.ds(g * 16, 16)] = jnp.maximum(xr + er, 0.0)
                return c

            lax.fori_loop(0, K, row, 0)

        def issue_scatter(b, ib):
            pltpu.async_copy(msg[b], acc.at[idx[ib].at[1]], ssc[b], add=True)

        def wait_scatter(b):
            pltpu.make_async_copy(
                msg[b], acc.at[idx[0].at[1]], ssc[b]).wait()

        for j in range(min(2 * NB, cpt)):
            issue_idx(j, j)

        def zero_row(j, c):
            for g in range(G):
                msg[0][j, pl.ds(g * 16, 16)] = jnp.zeros((16,), jnp.float32)
            return c

        lax.fori_loop(0, K, zero_row, 0)
        for i in range(nz_full):
            pltpu.sync_copy(
                msg[0], acc.at[pl.ds(pl.multiple_of(sid * B8 + i * K, 8), K)])
        if nz_tail:
            pltpu.sync_copy(
                msg[0].at[pl.ds(0, nz_tail)],
                acc.at[pl.ds(pl.multiple_of(sid * B8 + nz_full * K, 8),
                             nz_tail)])
        if REM:
            @pl.when(sid == NS - 1)
            def _():
                pltpu.sync_copy(msg[0].at[pl.ds(0, REM)],
                                acc.at[pl.ds(N - REM, REM)])

        for b in range(NB):
            wait_idx(b)
            issue_fetch(b, b, b)
        plsc.subcore_barrier()

        def ring(q, c):
            t0 = q * NI
            for j in range(NI):
                t = t0 + j
                b = j % NB
                wait_fetch(b)

                @pl.when(t >= NB)
                def _():
                    wait_scatter(b)

                compute(b)
                issue_scatter(b, j)
                tn = t + 2 * NB

                @pl.when(tn < cpt)
                def _():
                    issue_idx((j + 2 * NB) % NI, tn)

                tf = t + NB

                @pl.when(tf < cpt)
                def _():
                    wait_idx((j + NB) % NI)
                    issue_fetch(b, (j + NB) % NI, tf)
            return c

        lax.fori_loop(0, n_ring // NI, ring, 0)
        for t in range(n_ring, cpt):
            j = t % NI
            b = j % NB
            wait_fetch(b)
            if t >= NB:
                wait_scatter(b)
            compute(b)
            issue_scatter(b, j)
            tn = t + 2 * NB
            if tn < cpt:
                issue_idx((j + 2 * NB) % NI, tn)
            tf = t + NB
            if tf < cpt:
                wait_idx((j + NB) % NI)
                issue_fetch(b, (j + NB) % NI, tf)
        for b in range(min(NB, cpt)):
            wait_scatter(b)
        plsc.subcore_barrier()

        r0 = pl.multiple_of(sid * B8, 8)
        pltpu.sync_copy(
            acc.at[pl.ds(r0, B8)],
            out_hbm.at[pl.ds(pl.multiple_of(cid * N + sid * B8, 8), B8)],
        )
        if REM:
            @pl.when(sid == NS - 1)
            def _():
                pltpu.sync_copy(
                    acc.at[pl.ds(N - REM, REM)],
                    out_hbm.at[pl.ds(pl.multiple_of(cid * N + N - REM, 8),
                                     REM)],
                )

    return body(xh, eh, ei4)


def _mlp_bn_body(x_ref, ah_ref, w1_ref, b1_ref, w2_ref, b2_ref,
                 eps_ref, gamma_ref, beta_ref, out_ref):
    n = x_ref.shape[0]
    x = x_ref[...]
    aggr = jnp.concatenate([ah_ref[:n, :], ah_ref[n:, :]], axis=1)
    h = (1.0 + eps_ref[0, 0]) * x + aggr
    h1 = jnp.maximum(
        jnp.dot(h, w1_ref[...], preferred_element_type=jnp.float32)
        + b1_ref[...], 0.0)
    h2 = (jnp.dot(h1, w2_ref[...], preferred_element_type=jnp.float32)
          + b2_ref[...])
    mean = jnp.mean(h2, axis=0, keepdims=True)
    var = jnp.mean((h2 - mean) ** 2, axis=0, keepdims=True)
    hn = (h2 - mean) * lax.rsqrt(var + 1e-5) * gamma_ref[...] + beta_ref[...]
    out_ref[...] = jnp.maximum(hn + x, 0.0)


def _mlp_bn(x, aggr2, W1, b1, W2, b2, eps, gamma, beta):
    N, F = x.shape
    return pl.pallas_call(
        _mlp_bn_body,
        out_shape=jax.ShapeDtypeStruct((N, F), jnp.float32),
    )(x, aggr2, W1, b1.reshape(1, -1), W2, b2.reshape(1, -1),
      eps.reshape(1, 1), gamma.reshape(1, -1), beta.reshape(1, -1))


def kernel(x, edge_index, edge_attr, Wlin, blin, W1, b1, W2, b2,
           eps, gamma, beta):
    N, F = x.shape
    E = edge_index.shape[1]
    H = F // 2
    K = 40
    src = edge_index[0]
    dst = edge_index[1]
    srcr = src.reshape(E // K, K)
    dstr = dst.reshape(E // K, K)
    ei0 = jnp.stack([srcr, dstr], axis=1)
    ei1 = jnp.stack([srcr + N, dstr], axis=1)
    ei4 = jnp.concatenate([ei0, ei1], axis=0)

    eh = _edge_linear(edge_attr, Wlin, blin, H)
    xh = jnp.concatenate([x[:, :H], x[:, H:]], axis=0)
    aggr2 = _sc_aggregate(xh, eh, ei4, N, E, H, K, NB=3)
    return _mlp_bn(x, aggr2, W1, b1, W2, b2, eps, gamma, beta)


# speedup vs baseline: 3.0587x; 1.0047x over previous
import functools

import jax
import jax.numpy as jnp
import numpy as np
from jax import lax
from jax.experimental import pallas as pl
from jax.experimental.pallas import tpu as pltpu
from jax.experimental.pallas import tpu_sc as plsc


def _edge_linear_body(ea_ref, wle_ref, wlo_ref, ble_ref, blo_ref, out_ref):
    ea = ea_ref[...]
    ev = (jnp.dot(ea, wle_ref[0], preferred_element_type=jnp.float32)
          + ble_ref[0])
    od = (jnp.dot(ea, wlo_ref[0], preferred_element_type=jnp.float32)
          + blo_ref[0])
    evb = lax.bitcast_convert_type(ev.astype(jnp.bfloat16),
                                   jnp.uint16).astype(jnp.int32)
    odb = lax.bitcast_convert_type(od.astype(jnp.bfloat16),
                                   jnp.uint16).astype(jnp.int32)
    out_ref[...] = evb | (odb << 16)


def _edge_linear(edge_attr, Wlin, blin, H):
    E, D = edge_attr.shape
    W = H // 2
    BE = 1600
    nb = E // BE
    grid = (2, nb)
    w = np.arange(W)
    ev_half = 32 * (w // 16) + (w % 16)
    ev_cols = np.concatenate([ev_half, H + ev_half])
    od_cols = ev_cols + 16
    return pl.pallas_call(
        _edge_linear_body,
        grid=grid,
        in_specs=[
            pl.BlockSpec((BE, D), lambda c, i: (i, 0)),
            pl.BlockSpec((1, D, W), lambda c, i: (c, 0, 0)),
            pl.BlockSpec((1, D, W), lambda c, i: (c, 0, 0)),
            pl.BlockSpec((1, 1, W), lambda c, i: (c, 0, 0)),
            pl.BlockSpec((1, 1, W), lambda c, i: (c, 0, 0)),
        ],
        out_specs=pl.BlockSpec((BE, W), lambda c, i: (c * nb + i, 0)),
        out_shape=jax.ShapeDtypeStruct((2 * E, W), jnp.int32),
    )(edge_attr,
      Wlin[:, ev_cols].reshape(D, 2, W).transpose(1, 0, 2),
      Wlin[:, od_cols].reshape(D, 2, W).transpose(1, 0, 2),
      blin[ev_cols].reshape(1, 2, W).transpose(1, 0, 2),
      blin[od_cols].reshape(1, 2, W).transpose(1, 0, 2))


def _sc_aggregate(xh, eh, ei4, N, E, H, K, NB=3):
    NS = 16
    C = E // K
    cpt = C // NS
    NI = 3 * NB
    n_ring = (cpt // NI) * NI
    G = H // 16
    B8 = (N // NS) // 8 * 8
    REM = N - B8 * NS
    nz_full, nz_tail = B8 // K, B8 % K
    assert REM % 8 == 0 and REM <= K and nz_tail % 8 == 0

    mesh = plsc.VectorSubcoreMesh(core_axis_name="c", subcore_axis_name="s")

    W = H // 2
    scratch = (
        [pltpu.VMEM((2, K), jnp.int32) for _ in range(NI)]
        + [pltpu.VMEM((K, H), jnp.float32) for _ in range(NB)]
        + [pltpu.VMEM((K, W), jnp.int32) for _ in range(NB)]
        + [pltpu.VMEM((K, H), jnp.float32) for _ in range(NB)]
        + [pltpu.VMEM_SHARED((N, H), jnp.float32)]
        + [pltpu.SemaphoreType.DMA for _ in range(NI + 3 * NB)]
    )

    @functools.partial(
        pl.kernel,
        out_type=jax.ShapeDtypeStruct((2 * N, H), jnp.float32),
        mesh=mesh,
        scratch_types=scratch,
    )
    def body(xh_hbm, eh_hbm, ei_hbm, out_hbm, *refs):
        idx = refs[0:NI]
        rows = refs[NI:NI + NB]
        ebuf = refs[NI + NB:NI + 2 * NB]
        msg = refs[NI + 2 * NB:NI + 3 * NB]
        acc = refs[NI + 3 * NB]
        si = refs[NI + 3 * NB + 1:2 * NI + 3 * NB + 1]
        sg = refs[2 * NI + 3 * NB + 1:2 * NI + 4 * NB + 1]
        se = refs[2 * NI + 4 * NB + 1:2 * NI + 5 * NB + 1]
        ssc = refs[2 * NI + 5 * NB + 1:2 * NI + 6 * NB + 1]

        cid = lax.axis_index("c")
        sid = lax.axis_index("s")
        base_chunk = cid * C + sid * cpt

        def issue_idx(ib, t):
            pltpu.async_copy(ei_hbm.at[base_chunk + t], idx[ib], si[ib])

        def wait_idx(ib):
            pltpu.make_async_copy(
                ei_hbm.at[0], idx[ib], si[ib]).wait()

        def issue_fetch(b, ib, t):
            pltpu.async_copy(xh_hbm.at[idx[ib].at[0]], rows[b], sg[b])
            erow = pl.multiple_of((base_chunk + t) * K, 8)
            pltpu.async_copy(eh_hbm.at[pl.ds(erow, K)], ebuf[b], se[b])

        def wait_fetch(b):
            pltpu.make_async_copy(
                xh_hbm.at[pl.ds(0, K)], rows[b], sg[b]).wait()
            pltpu.make_async_copy(
                eh_hbm.at[pl.ds(0, K)], ebuf[b], se[b]).wait()

        himask = jnp.int32(-65536)

        def compute(b):
            def row(j, c):
                for g in range(G // 2):
                    ew = ebuf[b][j, pl.ds(g * 16, 16)]
                    lo = lax.bitcast_convert_type(ew << 16, jnp.float32)
                    hi = lax.bitcast_convert_type(ew & himask, jnp.float32)
                    xa = rows[b][j, pl.ds(g * 32, 16)]
                    xb = rows[b][j, pl.ds(g * 32 + 16, 16)]
                    msg[b][j, pl.ds(g * 32, 16)] = jnp.maximum(xa + lo, 0.0)
                    msg[b][j, pl.ds(g * 32 + 16, 16)] = jnp.maximum(
                        xb + hi, 0.0)
                return c

            lax.fori_loop(0, K, row, 0)

        def issue_scatter(b, ib):
            pltpu.async_copy(msg[b], acc.at[idx[ib].at[1]], ssc[b], add=True)

        def wait_scatter(b):
            pltpu.make_async_copy(
                msg[b], acc.at[idx[0].at[1]], ssc[b]).wait()

        for j in range(min(2 * NB, cpt)):
            issue_idx(j, j)

        def zero_row(j, c):
            for g in range(G):
                msg[0][j, pl.ds(g * 16, 16)] = jnp.zeros((16,), jnp.float32)
            return c

        lax.fori_loop(0, K, zero_row, 0)
        for i in range(nz_full):
            pltpu.sync_copy(
                msg[0], acc.at[pl.ds(pl.multiple_of(sid * B8 + i * K, 8), K)])
        if nz_tail:
            pltpu.sync_copy(
                msg[0].at[pl.ds(0, nz_tail)],
                acc.at[pl.ds(pl.multiple_of(sid * B8 + nz_full * K, 8),
                             nz_tail)])
        if REM:
            @pl.when(sid == NS - 1)
            def _():
                pltpu.sync_copy(msg[0].at[pl.ds(0, REM)],
                                acc.at[pl.ds(N - REM, REM)])

        for b in range(NB):
            wait_idx(b)
            issue_fetch(b, b, b)
        plsc.subcore_barrier()

        def ring(q, c):
            t0 = q * NI
            for j in range(NI):
                t = t0 + j
                b = j % NB
                wait_fetch(b)

                @pl.when(t >= NB)
                def _():
                    wait_scatter(b)

                compute(b)
                issue_scatter(b, j)
                tn = t + 2 * NB

                @pl.when(tn < cpt)
                def _():
                    issue_idx((j + 2 * NB) % NI, tn)

                tf = t + NB

                @pl.when(tf < cpt)
                def _():
                    wait_idx((j + NB) % NI)
                    issue_fetch(b, (j + NB) % NI, tf)
            return c

        lax.fori_loop(0, n_ring // NI, ring, 0)
        for t in range(n_ring, cpt):
            j = t % NI
            b = j % NB
            wait_fetch(b)
            if t >= NB:
                wait_scatter(b)
            compute(b)
            issue_scatter(b, j)
            tn = t + 2 * NB
            if tn < cpt:
                issue_idx((j + 2 * NB) % NI, tn)
            tf = t + NB
            if tf < cpt:
                wait_idx((j + NB) % NI)
                issue_fetch(b, (j + NB) % NI, tf)
        for b in range(min(NB, cpt)):
            wait_scatter(b)
        plsc.subcore_barrier()

        r0 = pl.multiple_of(sid * B8, 8)
        pltpu.sync_copy(
            acc.at[pl.ds(r0, B8)],
            out_hbm.at[pl.ds(pl.multiple_of(cid * N + sid * B8, 8), B8)],
        )
        if REM:
            @pl.when(sid == NS - 1)
            def _():
                pltpu.sync_copy(
                    acc.at[pl.ds(N - REM, REM)],
                    out_hbm.at[pl.ds(pl.multiple_of(cid * N + N - REM, 8),
                                     REM)],
                )

    return body(xh, eh, ei4)


def _mlp_bn_body(x_ref, ah_ref, w1_ref, b1_ref, w2_ref, b2_ref,
                 eps_ref, gamma_ref, beta_ref, out_ref):
    n = x_ref.shape[0]
    x = x_ref[...]
    aggr = jnp.concatenate([ah_ref[:n, :], ah_ref[n:, :]], axis=1)
    h = (1.0 + eps_ref[0, 0]) * x + aggr
    h1 = jnp.maximum(
        jnp.dot(h, w1_ref[...], preferred_element_type=jnp.float32)
        + b1_ref[...], 0.0)
    h2 = (jnp.dot(h1, w2_ref[...], preferred_element_type=jnp.float32)
          + b2_ref[...])
    mean = jnp.mean(h2, axis=0, keepdims=True)
    var = jnp.mean((h2 - mean) ** 2, axis=0, keepdims=True)
    hn = (h2 - mean) * lax.rsqrt(var + 1e-5) * gamma_ref[...] + beta_ref[...]
    out_ref[...] = jnp.maximum(hn + x, 0.0)


def _mlp_bn(x, aggr2, W1, b1, W2, b2, eps, gamma, beta):
    N, F = x.shape
    return pl.pallas_call(
        _mlp_bn_body,
        out_shape=jax.ShapeDtypeStruct((N, F), jnp.float32),
    )(x, aggr2, W1, b1.reshape(1, -1), W2, b2.reshape(1, -1),
      eps.reshape(1, 1), gamma.reshape(1, -1), beta.reshape(1, -1))


def kernel(x, edge_index, edge_attr, Wlin, blin, W1, b1, W2, b2,
           eps, gamma, beta):
    N, F = x.shape
    E = edge_index.shape[1]
    H = F // 2
    K = 40
    src = edge_index[0]
    dst = edge_index[1]
    srcr = src.reshape(E // K, K)
    dstr = dst.reshape(E // K, K)
    ei0 = jnp.stack([srcr, dstr], axis=1)
    ei1 = jnp.stack([srcr + N, dstr], axis=1)
    ei4 = jnp.concatenate([ei0, ei1], axis=0)

    eh = _edge_linear(edge_attr, Wlin, blin, H)
    xh = jnp.concatenate([x[:, :H], x[:, H:]], axis=0)
    aggr2 = _sc_aggregate(xh, eh, ei4, N, E, H, K, NB=3)
    return _mlp_bn(x, aggr2, W1, b1, W2, b2, eps, gamma, beta)
